# baseline (speedup 1.0000x reference)
import functools

import jax
import jax.numpy as jnp
from jax import lax
from jax.experimental import pallas as pl
from jax.experimental.pallas import tpu as pltpu
from jax.experimental.pallas import tpu_sc as plsc

VOCAB = 1000000
HIDDEN = 64
XROWS, XCOLS = 4096, 200
NC, NS = 2, 16
NW = NC * NS
XR_PER_W = XROWS // NW

SPLITS = ((0, 104), (104, 96))
CH_R = 4
N_CHUNKS = XR_PER_W // CH_R

HALF_V = 524288
BC = 16384
NB = HALF_V // BC
LAST_B = (VOCAB + BC - 1) // BC - 1


def _repack_body(a_ref, b_ref, o_ref):
    o_ref[...] = jnp.concatenate([a_ref[...].T, b_ref[...].T], axis=-1)


_repack = pl.pallas_call(
    _repack_body,
    grid=(NB,),
    in_specs=[
        pl.BlockSpec((HIDDEN, BC), lambda i: (0, i)),
        pl.BlockSpec((HIDDEN, BC), lambda i: (0, jnp.minimum(i + NB, LAST_B))),
    ],
    out_specs=pl.BlockSpec((BC, 128), lambda i: (i, 0)),
    out_shape=jax.ShapeDtypeStruct((HALF_V, 128), jnp.float32),
    compiler_params=pltpu.CompilerParams(vmem_limit_bytes=100 * 1024 * 1024),
)


def _emb_body(x_hbm, table_hbm, out_hbm, idx_all, rows_v, sem_g0, sem_g1,
              sem_s0, sem_s1):
    wid = lax.axis_index("s") * NC + lax.axis_index("c")
    xrow0 = wid * XR_PER_W
    sem_g = (sem_g0, sem_g1)
    sem_s = (sem_s0, sem_s1)

    pltpu.sync_copy(x_hbm.at[pl.ds(xrow0, XR_PER_W)], idx_all)

    def fire_gathers(c, b):
        for rr in range(CH_R):
            for off, ln in SPLITS:
                pltpu.async_copy(
                    table_hbm.at[idx_all.at[c * CH_R + rr, pl.ds(off, ln)]],
                    rows_v.at[b, rr, pl.ds(off, ln)],
                    sem_g[b],
                )

    def wait_gathers(b):
        for rr in range(CH_R):
            for off, ln in SPLITS:
                pltpu.make_async_copy(
                    table_hbm.at[idx_all.at[rr, pl.ds(off, ln)]],
                    rows_v.at[b, rr, pl.ds(off, ln)],
                    sem_g[b],
                ).wait()

    def start_store(c, b):
        pltpu.async_copy(
            rows_v.at[b],
            out_hbm.at[pl.ds(xrow0 + c * CH_R, CH_R), slice(None), pl.ds(0, HIDDEN)],
            sem_s[b],
        )

    def wait_store(b):
        pltpu.make_async_copy(
            rows_v.at[b],
            out_hbm.at[pl.ds(xrow0, CH_R), slice(None), pl.ds(0, HIDDEN)],
            sem_s[b],
        ).wait()

    def steady(c, b):
        wait_store(1 - b)
        fire_gathers(c + 1, 1 - b)
        wait_gathers(b)
        start_store(c, b)

    fire_gathers(0, 0)
    fire_gathers(1, 1)
    wait_gathers(0)
    start_store(0, 0)

    def chunk_pair(i, _):
        steady(2 * i + 1, 1)
        steady(2 * i + 2, 0)
        return ()

    lax.fori_loop(0, (N_CHUNKS - 2) // 2, chunk_pair, ())

    wait_store(0)
    wait_gathers(1)
    start_store(N_CHUNKS - 1, 1)
    wait_store(1)


_emb = functools.partial(
    pl.kernel,
    mesh=plsc.VectorSubcoreMesh(core_axis_name="c", subcore_axis_name="s"),
    out_type=jax.ShapeDtypeStruct((XROWS, XCOLS, 128), jnp.float32),
    scratch_types=[
        pltpu.VMEM((XR_PER_W, XCOLS), jnp.int32),
        pltpu.VMEM((2, CH_R, XCOLS, HIDDEN), jnp.float32),
        pltpu.SemaphoreType.DMA,
        pltpu.SemaphoreType.DMA,
        pltpu.SemaphoreType.DMA,
        pltpu.SemaphoreType.DMA,
    ],
    compiler_params=pltpu.CompilerParams(use_tc_tiling_on_sc=False),
)(_emb_body)


def kernel(x, table):
    packed = _repack(table.T, table.T)
    t64 = packed.reshape(2 * HALF_V, HIDDEN)
    xm = jnp.where(x < HALF_V, x * 2, (x - HALF_V) * 2 + 1)
    return _emb(xm, t64)[:, :, :HIDDEN]


# speedup vs baseline: 1.0083x; 1.0083x over previous
import functools

import jax
import jax.numpy as jnp
from jax import lax
from jax.experimental import pallas as pl
from jax.experimental.pallas import tpu as pltpu
from jax.experimental.pallas import tpu_sc as plsc

VOCAB = 1000000
HIDDEN = 64
XROWS, XCOLS = 4096, 200
NC, NS = 2, 16
NW = NC * NS
XR_PER_W = XROWS // NW

SPLITS = ((0, 104), (104, 96))
CH_R = 2
N_CHUNKS = XR_PER_W // CH_R

HALF_V = 524288
BC = 16384
NB = HALF_V // BC
LAST_B = (VOCAB + BC - 1) // BC - 1


def _repack_body(a_ref, b_ref, o_ref):
    o_ref[...] = jnp.concatenate([a_ref[...].T, b_ref[...].T], axis=-1)


_repack = pl.pallas_call(
    _repack_body,
    grid=(NB,),
    in_specs=[
        pl.BlockSpec((HIDDEN, BC), lambda i: (0, i)),
        pl.BlockSpec((HIDDEN, BC), lambda i: (0, jnp.minimum(i + NB, LAST_B))),
    ],
    out_specs=pl.BlockSpec((BC, 128), lambda i: (i, 0)),
    out_shape=jax.ShapeDtypeStruct((HALF_V, 128), jnp.float32),
    compiler_params=pltpu.CompilerParams(vmem_limit_bytes=100 * 1024 * 1024),
)


def _emb_body(x_hbm, table_hbm, out_hbm, idx_all, rows_v, sem_g0, sem_g1,
              sem_s0, sem_s1):
    wid = lax.axis_index("s") * NC + lax.axis_index("c")
    xrow0 = wid * XR_PER_W
    sem_g = (sem_g0, sem_g1)
    sem_s = (sem_s0, sem_s1)

    pltpu.sync_copy(x_hbm.at[pl.ds(xrow0, XR_PER_W)], idx_all)

    def fire_gathers(c, b):
        for rr in range(CH_R):
            for off, ln in SPLITS:
                pltpu.async_copy(
                    table_hbm.at[idx_all.at[c * CH_R + rr, pl.ds(off, ln)]],
                    rows_v.at[b, rr, pl.ds(off, ln)],
                    sem_g[b],
                )

    def wait_gathers(b):
        for rr in range(CH_R):
            for off, ln in SPLITS:
                pltpu.make_async_copy(
                    table_hbm.at[idx_all.at[rr, pl.ds(off, ln)]],
                    rows_v.at[b, rr, pl.ds(off, ln)],
                    sem_g[b],
                ).wait()

    def start_store(c, b):
        pltpu.async_copy(
            rows_v.at[b],
            out_hbm.at[pl.ds(xrow0 + c * CH_R, CH_R), slice(None), pl.ds(0, HIDDEN)],
            sem_s[b],
        )

    def wait_store(b):
        pltpu.make_async_copy(
            rows_v.at[b],
            out_hbm.at[pl.ds(xrow0, CH_R), slice(None), pl.ds(0, HIDDEN)],
            sem_s[b],
        ).wait()

    def steady(c, b):
        wait_store(1 - b)
        fire_gathers(c + 1, 1 - b)
        wait_gathers(b)
        start_store(c, b)

    fire_gathers(0, 0)
    fire_gathers(1, 1)
    wait_gathers(0)
    start_store(0, 0)

    def chunk_pair(i, _):
        steady(2 * i + 1, 1)
        steady(2 * i + 2, 0)
        return ()

    lax.fori_loop(0, (N_CHUNKS - 2) // 2, chunk_pair, ())

    wait_store(0)
    wait_gathers(1)
    start_store(N_CHUNKS - 1, 1)
    wait_store(1)


_emb = functools.partial(
    pl.kernel,
    mesh=plsc.VectorSubcoreMesh(core_axis_name="c", subcore_axis_name="s"),
    out_type=jax.ShapeDtypeStruct((XROWS, XCOLS, 128), jnp.float32),
    scratch_types=[
        pltpu.VMEM((XR_PER_W, XCOLS), jnp.int32),
        pltpu.VMEM((2, CH_R, XCOLS, HIDDEN), jnp.float32),
        pltpu.SemaphoreType.DMA,
        pltpu.SemaphoreType.DMA,
        pltpu.SemaphoreType.DMA,
        pltpu.SemaphoreType.DMA,
    ],
    compiler_params=pltpu.CompilerParams(use_tc_tiling_on_sc=False),
)(_emb_body)


def kernel(x, table):
    packed = _repack(table.T, table.T)
    t64 = packed.reshape(2 * HALF_V, HIDDEN)
    xm = jnp.where(x < HALF_V, x * 2, (x - HALF_V) * 2 + 1)
    return _emb(xm, t64)[:, :, :HIDDEN]


# speedup vs baseline: 1.0092x; 1.0009x over previous
import functools

import jax
import jax.numpy as jnp
from jax import lax
from jax.experimental import pallas as pl
from jax.experimental.pallas import tpu as pltpu
from jax.experimental.pallas import tpu_sc as plsc

VOCAB = 1000000
HIDDEN = 64
XROWS, XCOLS = 4096, 200
NC, NS = 2, 16
NW = NC * NS
XR_PER_W = XROWS // NW

SPLITS = ((0, 104), (104, 96))
CH_R = 2
N_CHUNKS = XR_PER_W // CH_R

HALF_V = 524288
BC = 16384
NB = HALF_V // BC
LAST_B = (VOCAB + BC - 1) // BC - 1


def _repack_body(a_ref, b_ref, o_ref):
    o_ref[...] = jnp.concatenate([a_ref[...].T, b_ref[...].T], axis=-1)


_repack = pl.pallas_call(
    _repack_body,
    grid=(NB,),
    in_specs=[
        pl.BlockSpec((HIDDEN, BC), lambda i: (0, i)),
        pl.BlockSpec((HIDDEN, BC), lambda i: (0, jnp.minimum(i + NB, LAST_B))),
    ],
    out_specs=pl.BlockSpec((BC, 128), lambda i: (i, 0)),
    out_shape=jax.ShapeDtypeStruct((HALF_V, 128), jnp.float32),
    compiler_params=pltpu.CompilerParams(vmem_limit_bytes=100 * 1024 * 1024),
)


def _emb_body(x_hbm, table_hbm, out_hbm, idx_all, rows_v, sem_g0, sem_g1,
              sem_g2, sem_s0, sem_s1, sem_s2):
    wid = lax.axis_index("s") * NC + lax.axis_index("c")
    xrow0 = wid * XR_PER_W
    sem_g = (sem_g0, sem_g1, sem_g2)
    sem_s = (sem_s0, sem_s1, sem_s2)

    pltpu.sync_copy(x_hbm.at[pl.ds(xrow0, XR_PER_W)], idx_all)

    def fire_gathers(c, b):
        for rr in range(CH_R):
            for off, ln in SPLITS:
                pltpu.async_copy(
                    table_hbm.at[idx_all.at[c * CH_R + rr, pl.ds(off, ln)]],
                    rows_v.at[b, rr, pl.ds(off, ln)],
                    sem_g[b],
                )

    def wait_gathers(b):
        for rr in range(CH_R):
            for off, ln in SPLITS:
                pltpu.make_async_copy(
                    table_hbm.at[idx_all.at[rr, pl.ds(off, ln)]],
                    rows_v.at[b, rr, pl.ds(off, ln)],
                    sem_g[b],
                ).wait()

    def start_store(c, b):
        pltpu.async_copy(
            rows_v.at[b],
            out_hbm.at[pl.ds(xrow0 + c * CH_R, CH_R), slice(None), pl.ds(0, HIDDEN)],
            sem_s[b],
        )

    def wait_store(b):
        pltpu.make_async_copy(
            rows_v.at[b],
            out_hbm.at[pl.ds(xrow0, CH_R), slice(None), pl.ds(0, HIDDEN)],
            sem_s[b],
        ).wait()

    def steady(c, b_cur, b_next):
        wait_store(b_next)
        fire_gathers(c + 2, b_next)
        wait_gathers(b_cur)
        start_store(c, b_cur)

    fire_gathers(0, 0)
    fire_gathers(1, 1)
    fire_gathers(2, 2)
    wait_gathers(0)
    start_store(0, 0)
    wait_store(0)
    fire_gathers(3, 0)
    wait_gathers(1)
    start_store(1, 1)

    def chunk_triple(i, _):
        c0 = 2 + 3 * i
        steady(c0, 2, 1)
        steady(c0 + 1, 0, 2)
        steady(c0 + 2, 1, 0)
        return ()

    lax.fori_loop(0, (N_CHUNKS - 4) // 3, chunk_triple, ())

    wait_store(1)
    wait_gathers(2)
    start_store(N_CHUNKS - 2, 2)
    wait_gathers(0)
    start_store(N_CHUNKS - 1, 0)
    wait_store(2)
    wait_store(0)


_emb = functools.partial(
    pl.kernel,
    mesh=plsc.VectorSubcoreMesh(core_axis_name="c", subcore_axis_name="s"),
    out_type=jax.ShapeDtypeStruct((XROWS, XCOLS, 128), jnp.float32),
    scratch_types=[
        pltpu.VMEM((XR_PER_W, XCOLS), jnp.int32),
        pltpu.VMEM((3, CH_R, XCOLS, HIDDEN), jnp.float32),
        pltpu.SemaphoreType.DMA,
        pltpu.SemaphoreType.DMA,
        pltpu.SemaphoreType.DMA,
        pltpu.SemaphoreType.DMA,
        pltpu.SemaphoreType.DMA,
        pltpu.SemaphoreType.DMA,
    ],
    compiler_params=pltpu.CompilerParams(use_tc_tiling_on_sc=False),
)(_emb_body)


def kernel(x, table):
    packed = _repack(table.T, table.T)
    t64 = packed.reshape(2 * HALF_V, HIDDEN)
    xm = jnp.where(x < HALF_V, x * 2, (x - HALF_V) * 2 + 1)
    return _emb(xm, t64)[:, :, :HIDDEN]
